# baseline (speedup 1.0000x reference)
import functools

import jax
import jax.numpy as jnp
from jax import lax
from jax.experimental import pallas as pl
from jax.experimental.pallas import tpu as pltpu
from jax.experimental.pallas import tpu_sc as plsc

N = 10000
E = 320000
INF = 128
HID = 128
OUT = 128

NP_ = 10240
EP_ = 327680
PAD_IDX = 10100

NB = 1280
EB = 1024


def _k1_body(h_ref, c_ref, v_ref, wa_ref, wb_ref, wv1_ref, bv1_ref,
             wv2_ref, bv2_ref, ta_ref, tb_ref, vmv_ref):
    h = h_ref[...]
    b = h.shape[0]
    cvpad = jnp.concatenate([c_ref[...], v_ref[...],
                             jnp.zeros((b, 124), jnp.float32)], axis=1)
    ta_ref[...] = jnp.concatenate(
        [jnp.dot(h, wa_ref[...], preferred_element_type=jnp.float32), cvpad],
        axis=1)
    tb_ref[...] = jnp.concatenate(
        [jnp.dot(h, wb_ref[...], preferred_element_type=jnp.float32), cvpad],
        axis=1)
    m = jnp.maximum(jnp.dot(h, wv1_ref[...],
                            preferred_element_type=jnp.float32) + bv1_ref[...], 0.0)
    vm = jnp.dot(m, wv2_ref[...], preferred_element_type=jnp.float32) + bv2_ref[...]
    vmv_ref[...] = vm * v_ref[...]


def _node_precompute(h_p, c_p, v_p, wa, wb, wv1, bv1, wv2, bv2):
    grid = NP_ // NB
    return pl.pallas_call(
        _k1_body,
        grid=(grid,),
        in_specs=[
            pl.BlockSpec((NB, INF), lambda i: (i, 0)),
            pl.BlockSpec((NB, 2), lambda i: (i, 0)),
            pl.BlockSpec((NB, 2), lambda i: (i, 0)),
            pl.BlockSpec((INF, HID), lambda i: (0, 0)),
            pl.BlockSpec((INF, HID), lambda i: (0, 0)),
            pl.BlockSpec((INF, HID), lambda i: (0, 0)),
            pl.BlockSpec((HID,), lambda i: (0,)),
            pl.BlockSpec((HID, 1), lambda i: (0, 0)),
            pl.BlockSpec((1,), lambda i: (0,)),
        ],
        out_specs=[
            pl.BlockSpec((NB, 2 * HID), lambda i: (i, 0)),
            pl.BlockSpec((NB, 2 * HID), lambda i: (i, 0)),
            pl.BlockSpec((NB, 2), lambda i: (i, 0)),
        ],
        out_shape=[
            jax.ShapeDtypeStruct((NP_, 2 * HID), jnp.float32),
            jax.ShapeDtypeStruct((NP_, 2 * HID), jnp.float32),
            jax.ShapeDtypeStruct((NP_, 2), jnp.float32),
        ],
    )(h_p, c_p, v_p, wa, wb, wv1, bv1, wv2, bv2)


def _pack_bf16(t):
    u0 = jax.lax.bitcast_convert_type(
        t[:, 0:HID].astype(jnp.bfloat16), jnp.uint16).astype(jnp.uint32)
    u1 = jax.lax.bitcast_convert_type(
        t[:, HID:2 * HID].astype(jnp.bfloat16), jnp.uint16).astype(jnp.uint32)
    return jax.lax.bitcast_convert_type(u0 | (u1 << 16), jnp.int32)


NWORK = 32
CHUNK = 64
NCHUNK = EP_ // CHUNK
CPW = NCHUNK // NWORK


def _sc_gather(ta, tb, row2d, col2d):
    nch = row2d.shape[0]
    cpw = nch // NWORK
    ep = nch * CHUNK
    mesh = plsc.VectorSubcoreMesh(core_axis_name="c", subcore_axis_name="s")

    @functools.partial(
        pl.kernel,
        mesh=mesh,
        out_type=[
            jax.ShapeDtypeStruct((ep, HID), jnp.int32),
            jax.ShapeDtypeStruct((ep, HID), jnp.int32),
        ],
        scratch_types=[
            pltpu.VMEM((cpw, CHUNK), jnp.int32),
            pltpu.VMEM((cpw, CHUNK), jnp.int32),
            pltpu.VMEM((CHUNK, HID), jnp.int32),
            pltpu.VMEM((CHUNK, HID), jnp.int32),
            pltpu.VMEM((CHUNK, HID), jnp.int32),
            pltpu.VMEM((CHUNK, HID), jnp.int32),
            pltpu.SemaphoreType.DMA,
            pltpu.SemaphoreType.DMA,
        ],
    )
    def k2(ta_hbm, tb_hbm, row_hbm, col_hbm, ga_hbm, gb_hbm,
           idxr, idxc, bufa0, bufb0, bufa1, bufb1, gsem, wsem):
        wid = lax.axis_index("s") * 2 + lax.axis_index("c")
        c0 = wid * cpw
        pltpu.sync_copy(row_hbm.at[pl.ds(c0, cpw)], idxr)
        pltpu.sync_copy(col_hbm.at[pl.ds(c0, cpw)], idxc)

        def gather(j, ba, bb):
            cp_a = pltpu.async_copy(ta_hbm.at[idxr.at[j]], ba, gsem)
            cp_b = pltpu.async_copy(tb_hbm.at[idxc.at[j]], bb, gsem)
            return cp_a, cp_b

        def write(j, ba, bb):
            base = (c0 + j) * CHUNK
            wa_ = pltpu.async_copy(ba, ga_hbm.at[pl.ds(base, CHUNK)], wsem)
            wb_ = pltpu.async_copy(bb, gb_hbm.at[pl.ds(base, CHUNK)], wsem)
            return wa_, wb_

        def body(t, carry):
            j0 = t * 2
            g0a, g0b = gather(j0, bufa0, bufb0)
            g1a, g1b = gather(j0 + 1, bufa1, bufb1)
            g0a.wait()
            g0b.wait()
            w0a, w0b = write(j0, bufa0, bufb0)
            g1a.wait()
            g1b.wait()
            w1a, w1b = write(j0 + 1, bufa1, bufb1)
            w0a.wait()
            w0b.wait()
            w1a.wait()
            w1b.wait()
            return carry

        lax.fori_loop(0, cpw // 2, body, 0)

    return k2(ta, tb, row2d, col2d)


RPS = NP_ // 16


def _sc_scatter(row2d, feat, small, initf, inits):
    nch = row2d.shape[0]
    cps = nch // 16
    mesh = plsc.VectorSubcoreMesh(core_axis_name="c", subcore_axis_name="s")

    @functools.partial(
        pl.kernel,
        mesh=mesh,
        out_type=[
            jax.ShapeDtypeStruct((NP_, HID), jnp.float32),
            jax.ShapeDtypeStruct((NP_, HID), jnp.float32),
        ],
        scratch_types=[
            pltpu.VMEM((1, CHUNK), jnp.int32),
            pltpu.VMEM((1, CHUNK), jnp.int32),
            pltpu.VMEM((CHUNK, HID), jnp.float32),
            pltpu.VMEM((CHUNK, HID), jnp.float32),
            pltpu.VMEM_SHARED((NP_, HID), jnp.float32),
            pltpu.SemaphoreType.DMA,
        ],
    )
    def k4(row_hbm, feat_hbm, small_hbm, initf_hbm, inits_hbm,
           outf_hbm, outs_hbm, idx0, idx1, buf0, buf1, shp, lsem):
        c = lax.axis_index("c")
        s = lax.axis_index("s")
        r0 = s * RPS

        @pl.when(c == 0)
        def _():
            pltpu.sync_copy(initf_hbm.at[pl.ds(r0, RPS)],
                            shp.at[pl.ds(r0, RPS)])

        @pl.when(c == 1)
        def _():
            pltpu.sync_copy(inits_hbm.at[pl.ds(r0, RPS)],
                            shp.at[pl.ds(r0, RPS)])

        plsc.subcore_barrier()

        def mk_body(src_hbm):
            def load(j, ib, b):
                chunk = s * cps + j
                ci = pltpu.async_copy(row_hbm.at[pl.ds(chunk, 1)], ib, lsem)
                cp = pltpu.async_copy(
                    src_hbm.at[pl.ds(chunk * CHUNK, CHUNK)], b, lsem)
                return ci, cp

            def body(t, carry):
                j0 = t * 2
                i0, l0 = load(j0, idx0, buf0)
                i1, l1 = load(j0 + 1, idx1, buf1)
                i0.wait()
                l0.wait()
                pltpu.sync_copy(buf0, shp.at[idx0.at[0]], add=True)
                i1.wait()
                l1.wait()
                pltpu.sync_copy(buf1, shp.at[idx1.at[0]], add=True)
                return carry

            lax.fori_loop(0, cps // 2, body, 0)

        @pl.when(c == 0)
        def _():
            mk_body(feat_hbm)

        @pl.when(c == 1)
        def _():
            mk_body(small_hbm)

        plsc.subcore_barrier()

        @pl.when(c == 0)
        def _():
            pltpu.sync_copy(shp.at[pl.ds(r0, RPS)], outf_hbm.at[pl.ds(r0, RPS)])

        @pl.when(c == 1)
        def _():
            pltpu.sync_copy(shp.at[pl.ds(r0, RPS)], outs_hbm.at[pl.ds(r0, RPS)])

    return k4(row2d, feat, small, initf, inits)


def _k3_body(g2a_ref, g2b_ref, wx_ref, we2_ref, be2_ref, wc1_ref, bc1_ref,
             wc2_ref, feat_ref, small_ref):
    wa_ = g2a_ref[...]
    wb_ = g2b_ref[...]
    himask = jnp.int32(-65536)
    ga = jax.lax.bitcast_convert_type(wa_ << 16, jnp.float32)
    gb = jax.lax.bitcast_convert_type(wb_ << 16, jnp.float32)
    cva = jax.lax.bitcast_convert_type(wa_[:, 0:4] & himask, jnp.float32)
    cvb = jax.lax.bitcast_convert_type(wb_[:, 0:4] & himask, jnp.float32)
    wx = wx_ref[...]
    cr = cva[:, 0:2]
    cc = cvb[:, 0:2]
    dv = cva[:, 2:4] - cvb[:, 2:4]
    d = cr - cc
    d2 = jnp.sum(d * d, axis=1, keepdims=True)
    p = (ga + gb + d2 * wx[4:5, :] + wx[7:8, :])
    x = (cr[:, 0:1] * wx[0:1, :] + cc[:, 0:1] * wx[2:3, :]
         + dv[:, 0:1] * wx[5:6, :])
    y = (cr[:, 1:2] * wx[1:2, :] + cc[:, 1:2] * wx[3:4, :]
         + dv[:, 1:2] * wx[6:7, :])
    m = jnp.concatenate([
        jnp.maximum(p + x + y, 0.0),
        jnp.maximum(p - x - y, 0.0),
        jnp.maximum(p - x + y, 0.0),
        jnp.maximum(p + x - y, 0.0),
    ], axis=0)
    r = jnp.maximum(jnp.dot(m, we2_ref[...],
                            preferred_element_type=jnp.float32) + be2_ref[...], 0.0)
    b = ga.shape[0]
    ef = 0.25 * (r[0:b] + r[b:2 * b] + r[2 * b:3 * b] + r[3 * b:4 * b])
    feat_ref[...] = ef
    cm = jnp.dot(
        jnp.maximum(jnp.dot(ef, wc1_ref[...],
                            preferred_element_type=jnp.float32) + bc1_ref[...], 0.0),
        wc2_ref[...], preferred_element_type=jnp.float32)
    trans = jnp.clip(d * cm, -100.0, 100.0)
    small_ref[...] = jnp.concatenate(
        [trans, jnp.ones((b, 1), jnp.float32),
         jnp.zeros((b, HID - 3), jnp.float32)], axis=1)


def _edge_mlp(g2a, g2b, wx, we2, be2, wc1, bc1, wc2):
    ep = g2a.shape[0]
    grid = ep // EB
    return pl.pallas_call(
        _k3_body,
        grid=(grid,),
        in_specs=[
            pl.BlockSpec((EB, HID), lambda i: (i, 0)),
            pl.BlockSpec((EB, HID), lambda i: (i, 0)),
            pl.BlockSpec((8, HID), lambda i: (0, 0)),
            pl.BlockSpec((HID, HID), lambda i: (0, 0)),
            pl.BlockSpec((HID,), lambda i: (0,)),
            pl.BlockSpec((HID, HID), lambda i: (0, 0)),
            pl.BlockSpec((HID,), lambda i: (0,)),
            pl.BlockSpec((HID, 2), lambda i: (0, 0)),
        ],
        out_specs=[
            pl.BlockSpec((EB, HID), lambda i: (i, 0)),
            pl.BlockSpec((EB, HID), lambda i: (i, 0)),
        ],
        out_shape=[
            jax.ShapeDtypeStruct((ep, HID), jnp.float32),
            jax.ShapeDtypeStruct((ep, HID), jnp.float32),
        ],
    )(g2a, g2b, wx, we2, be2, wc1, bc1, wc2)


def _k5_body(h_ref, c_ref, vmv_ref, aggf_ref, aggs_ref, wn1a_ref, wn1b_ref,
             bn1_ref, wn2_ref, bn2_ref, hout_ref, cout_ref):
    h = h_ref[...]
    agg = aggf_ref[...]
    s = aggs_ref[...]
    cnt = jnp.maximum(s[:, 2:3], 1.0)
    cout_ref[...] = c_ref[...] + s[:, 0:2] / cnt + vmv_ref[...]
    t = jnp.maximum(
        jnp.dot(h, wn1a_ref[...], preferred_element_type=jnp.float32)
        + jnp.dot(agg, wn1b_ref[...], preferred_element_type=jnp.float32)
        + bn1_ref[...], 0.0)
    hout_ref[...] = (h + jnp.dot(t, wn2_ref[...],
                                 preferred_element_type=jnp.float32) + bn2_ref[...])


def _node_finalize(h_p, c_p, vmv, aggf, aggs, wn1a, wn1b, bn1, wn2, bn2):
    grid = NP_ // NB
    return pl.pallas_call(
        _k5_body,
        grid=(grid,),
        in_specs=[
            pl.BlockSpec((NB, INF), lambda i: (i, 0)),
            pl.BlockSpec((NB, 2), lambda i: (i, 0)),
            pl.BlockSpec((NB, 2), lambda i: (i, 0)),
            pl.BlockSpec((NB, HID), lambda i: (i, 0)),
            pl.BlockSpec((NB, HID), lambda i: (i, 0)),
            pl.BlockSpec((INF, HID), lambda i: (0, 0)),
            pl.BlockSpec((HID, HID), lambda i: (0, 0)),
            pl.BlockSpec((HID,), lambda i: (0,)),
            pl.BlockSpec((HID, OUT), lambda i: (0, 0)),
            pl.BlockSpec((OUT,), lambda i: (0,)),
        ],
        out_specs=[
            pl.BlockSpec((NB, OUT), lambda i: (i, 0)),
            pl.BlockSpec((NB, 2), lambda i: (i, 0)),
        ],
        out_shape=[
            jax.ShapeDtypeStruct((NP_, OUT), jnp.float32),
            jax.ShapeDtypeStruct((NP_, 2), jnp.float32),
        ],
    )(h_p, c_p, vmv, aggf, aggs, wn1a, wn1b, bn1, wn2, bn2)


def kernel(h, edge_index, coord, vel, We1, be1, We2, be2, Wn1, bn1, Wn2,
           bn2, Wc1, bc1, Wc2, Wv1, bv1, Wv2, bv2):
    h_p = jnp.pad(h, ((0, NP_ - N), (0, 0)))
    c_p = jnp.pad(coord, ((0, NP_ - N), (0, 0)))
    v_p = jnp.pad(vel, ((0, NP_ - N), (0, 0)))
    row = jnp.pad(edge_index[0], (0, EP_ - E), constant_values=PAD_IDX)
    col = jnp.pad(edge_index[1], (0, EP_ - E), constant_values=PAD_IDX)

    wa = We1[0:INF]
    wb = We1[INF:2 * INF]
    wx = jnp.concatenate([We1[2 * INF:2 * INF + 7], be1[None, :]], axis=0)

    ta, tb, vmv = _node_precompute(h_p, c_p, v_p, wa, wb, Wv1, bv1, Wv2, bv2)

    row2d = row.reshape(NCHUNK, CHUNK)
    col2d = col.reshape(NCHUNK, CHUNK)
    ta_p = _pack_bf16(ta)
    tb_p = _pack_bf16(tb)
    zf = jnp.zeros((NP_, HID), jnp.float32)

    nslice = 10
    hc = NCHUNK // nslice
    aggf, aggs = zf, zf
    for hidx in range(nslice):
        r2 = row2d[hidx * hc:(hidx + 1) * hc]
        c2 = col2d[hidx * hc:(hidx + 1) * hc]
        g2a, g2b = _sc_gather(ta_p, tb_p, r2, c2)
        feat, small = _edge_mlp(g2a, g2b, wx, We2, be2, Wc1, bc1, Wc2)
        aggf, aggs = _sc_scatter(r2, feat, small, aggf, aggs)

    hout, cout = _node_finalize(h_p, c_p, vmv, aggf, aggs, Wn1[0:INF],
                                Wn1[INF:], bn1, Wn2, bn2)
    return hout[:N], cout[:N]


# speedup vs baseline: 1.0281x; 1.0281x over previous
import functools

import jax
import jax.numpy as jnp
from jax import lax
from jax.experimental import pallas as pl
from jax.experimental.pallas import tpu as pltpu
from jax.experimental.pallas import tpu_sc as plsc

N = 10000
E = 320000
INF = 128
HID = 128
OUT = 128

NP_ = 10240
EP_ = 327680
PAD_IDX = 10100

NB = 1280
EB = 1024


def _k1_body(h_ref, c_ref, v_ref, wa_ref, wb_ref, wv1_ref, bv1_ref,
             wv2_ref, bv2_ref, ta_ref, tb_ref, vmv_ref):
    h = h_ref[...]
    b = h.shape[0]
    cvpad = jnp.concatenate([c_ref[...], v_ref[...],
                             jnp.zeros((b, 124), jnp.float32)], axis=1)
    ta_ref[...] = jnp.concatenate(
        [jnp.dot(h, wa_ref[...], preferred_element_type=jnp.float32), cvpad],
        axis=1)
    tb_ref[...] = jnp.concatenate(
        [jnp.dot(h, wb_ref[...], preferred_element_type=jnp.float32), cvpad],
        axis=1)
    m = jnp.maximum(jnp.dot(h, wv1_ref[...],
                            preferred_element_type=jnp.float32) + bv1_ref[...], 0.0)
    vm = jnp.dot(m, wv2_ref[...], preferred_element_type=jnp.float32) + bv2_ref[...]
    vmv_ref[...] = vm * v_ref[...]


def _node_precompute(h_p, c_p, v_p, wa, wb, wv1, bv1, wv2, bv2):
    grid = NP_ // NB
    return pl.pallas_call(
        _k1_body,
        grid=(grid,),
        in_specs=[
            pl.BlockSpec((NB, INF), lambda i: (i, 0)),
            pl.BlockSpec((NB, 2), lambda i: (i, 0)),
            pl.BlockSpec((NB, 2), lambda i: (i, 0)),
            pl.BlockSpec((INF, HID), lambda i: (0, 0)),
            pl.BlockSpec((INF, HID), lambda i: (0, 0)),
            pl.BlockSpec((INF, HID), lambda i: (0, 0)),
            pl.BlockSpec((HID,), lambda i: (0,)),
            pl.BlockSpec((HID, 1), lambda i: (0, 0)),
            pl.BlockSpec((1,), lambda i: (0,)),
        ],
        out_specs=[
            pl.BlockSpec((NB, 2 * HID), lambda i: (i, 0)),
            pl.BlockSpec((NB, 2 * HID), lambda i: (i, 0)),
            pl.BlockSpec((NB, 2), lambda i: (i, 0)),
        ],
        out_shape=[
            jax.ShapeDtypeStruct((NP_, 2 * HID), jnp.float32),
            jax.ShapeDtypeStruct((NP_, 2 * HID), jnp.float32),
            jax.ShapeDtypeStruct((NP_, 2), jnp.float32),
        ],
    )(h_p, c_p, v_p, wa, wb, wv1, bv1, wv2, bv2)


def _pack_bf16(t):
    u0 = jax.lax.bitcast_convert_type(
        t[:, 0:HID].astype(jnp.bfloat16), jnp.uint16).astype(jnp.uint32)
    u1 = jax.lax.bitcast_convert_type(
        t[:, HID:2 * HID].astype(jnp.bfloat16), jnp.uint16).astype(jnp.uint32)
    return jax.lax.bitcast_convert_type(u0 | (u1 << 16), jnp.int32)


NWORK = 32
CHUNK = 64
NCHUNK = EP_ // CHUNK
CPW = NCHUNK // NWORK


def _sc_gather(ta, tb, row2d, col2d):
    nch = row2d.shape[0]
    cpw = nch // NWORK
    ep = nch * CHUNK
    mesh = plsc.VectorSubcoreMesh(core_axis_name="c", subcore_axis_name="s")

    @functools.partial(
        pl.kernel,
        mesh=mesh,
        out_type=[
            jax.ShapeDtypeStruct((ep, HID), jnp.int32),
            jax.ShapeDtypeStruct((ep, HID), jnp.int32),
        ],
        scratch_types=[
            pltpu.VMEM((cpw, CHUNK), jnp.int32),
            pltpu.VMEM((cpw, CHUNK), jnp.int32),
            pltpu.VMEM((CHUNK, HID), jnp.int32),
            pltpu.VMEM((CHUNK, HID), jnp.int32),
            pltpu.VMEM((CHUNK, HID), jnp.int32),
            pltpu.VMEM((CHUNK, HID), jnp.int32),
            pltpu.SemaphoreType.DMA,
            pltpu.SemaphoreType.DMA,
        ],
    )
    def k2(ta_hbm, tb_hbm, row_hbm, col_hbm, ga_hbm, gb_hbm,
           idxr, idxc, bufa0, bufb0, bufa1, bufb1, gsem, wsem):
        wid = lax.axis_index("s") * 2 + lax.axis_index("c")
        c0 = wid * cpw
        pltpu.sync_copy(row_hbm.at[pl.ds(c0, cpw)], idxr)
        pltpu.sync_copy(col_hbm.at[pl.ds(c0, cpw)], idxc)

        def gather(j, ba, bb):
            cp_a = pltpu.async_copy(ta_hbm.at[idxr.at[j]], ba, gsem)
            cp_b = pltpu.async_copy(tb_hbm.at[idxc.at[j]], bb, gsem)
            return cp_a, cp_b

        def write(j, ba, bb):
            base = (c0 + j) * CHUNK
            wa_ = pltpu.async_copy(ba, ga_hbm.at[pl.ds(base, CHUNK)], wsem)
            wb_ = pltpu.async_copy(bb, gb_hbm.at[pl.ds(base, CHUNK)], wsem)
            return wa_, wb_

        def body(t, carry):
            j0 = t * 2
            g0a, g0b = gather(j0, bufa0, bufb0)
            g1a, g1b = gather(j0 + 1, bufa1, bufb1)
            g0a.wait()
            g0b.wait()
            w0a, w0b = write(j0, bufa0, bufb0)
            g1a.wait()
            g1b.wait()
            w1a, w1b = write(j0 + 1, bufa1, bufb1)
            w0a.wait()
            w0b.wait()
            w1a.wait()
            w1b.wait()
            return carry

        lax.fori_loop(0, cpw // 2, body, 0)

    return k2(ta, tb, row2d, col2d)


RPS = NP_ // 16


def _sc_scatter(row2d, feat, small, initf, inits):
    nch = row2d.shape[0]
    cps = nch // 16
    mesh = plsc.VectorSubcoreMesh(core_axis_name="c", subcore_axis_name="s")

    @functools.partial(
        pl.kernel,
        mesh=mesh,
        out_type=[
            jax.ShapeDtypeStruct((NP_, HID), jnp.float32),
            jax.ShapeDtypeStruct((NP_, HID), jnp.float32),
        ],
        scratch_types=[
            pltpu.VMEM((1, CHUNK), jnp.int32),
            pltpu.VMEM((1, CHUNK), jnp.int32),
            pltpu.VMEM((CHUNK, HID), jnp.float32),
            pltpu.VMEM((CHUNK, HID), jnp.float32),
            pltpu.VMEM_SHARED((NP_, HID), jnp.float32),
            pltpu.SemaphoreType.DMA,
        ],
    )
    def k4(row_hbm, feat_hbm, small_hbm, initf_hbm, inits_hbm,
           outf_hbm, outs_hbm, idx0, idx1, buf0, buf1, shp, lsem):
        c = lax.axis_index("c")
        s = lax.axis_index("s")
        r0 = s * RPS

        @pl.when(c == 0)
        def _():
            pltpu.sync_copy(initf_hbm.at[pl.ds(r0, RPS)],
                            shp.at[pl.ds(r0, RPS)])

        @pl.when(c == 1)
        def _():
            pltpu.sync_copy(inits_hbm.at[pl.ds(r0, RPS)],
                            shp.at[pl.ds(r0, RPS)])

        plsc.subcore_barrier()

        def mk_body(src_hbm):
            def load(j, ib, b):
                chunk = s * cps + j
                ci = pltpu.async_copy(row_hbm.at[pl.ds(chunk, 1)], ib, lsem)
                cp = pltpu.async_copy(
                    src_hbm.at[pl.ds(chunk * CHUNK, CHUNK)], b, lsem)
                return ci, cp

            def body(t, carry):
                j0 = t * 2
                i0, l0 = load(j0, idx0, buf0)
                i1, l1 = load(j0 + 1, idx1, buf1)
                i0.wait()
                l0.wait()
                pltpu.sync_copy(buf0, shp.at[idx0.at[0]], add=True)
                i1.wait()
                l1.wait()
                pltpu.sync_copy(buf1, shp.at[idx1.at[0]], add=True)
                return carry

            lax.fori_loop(0, cps // 2, body, 0)

        @pl.when(c == 0)
        def _():
            mk_body(feat_hbm)

        @pl.when(c == 1)
        def _():
            mk_body(small_hbm)

        plsc.subcore_barrier()

        @pl.when(c == 0)
        def _():
            pltpu.sync_copy(shp.at[pl.ds(r0, RPS)], outf_hbm.at[pl.ds(r0, RPS)])

        @pl.when(c == 1)
        def _():
            pltpu.sync_copy(shp.at[pl.ds(r0, RPS)], outs_hbm.at[pl.ds(r0, RPS)])

    return k4(row2d, feat, small, initf, inits)


def _k3_body(g2a_ref, g2b_ref, wx_ref, we2_ref, be2_ref, wc1_ref, bc1_ref,
             wc2_ref, feat_ref, small_ref):
    wa_ = g2a_ref[...]
    wb_ = g2b_ref[...]
    himask = jnp.int32(-65536)
    ga = jax.lax.bitcast_convert_type(wa_ << 16, jnp.float32)
    gb = jax.lax.bitcast_convert_type(wb_ << 16, jnp.float32)
    cva = jax.lax.bitcast_convert_type(wa_[:, 0:4] & himask, jnp.float32)
    cvb = jax.lax.bitcast_convert_type(wb_[:, 0:4] & himask, jnp.float32)
    wx = wx_ref[...]
    cr = cva[:, 0:2]
    cc = cvb[:, 0:2]
    dv = cva[:, 2:4] - cvb[:, 2:4]
    d = cr - cc
    d2 = jnp.sum(d * d, axis=1, keepdims=True)
    p = (ga + gb + d2 * wx[4:5, :] + wx[7:8, :])
    x = (cr[:, 0:1] * wx[0:1, :] + cc[:, 0:1] * wx[2:3, :]
         + dv[:, 0:1] * wx[5:6, :])
    y = (cr[:, 1:2] * wx[1:2, :] + cc[:, 1:2] * wx[3:4, :]
         + dv[:, 1:2] * wx[6:7, :])
    m = jnp.concatenate([
        jnp.maximum(p + x + y, 0.0),
        jnp.maximum(p - x - y, 0.0),
        jnp.maximum(p - x + y, 0.0),
        jnp.maximum(p + x - y, 0.0),
    ], axis=0)
    r = jnp.maximum(jnp.dot(m, we2_ref[...],
                            preferred_element_type=jnp.float32) + be2_ref[...], 0.0)
    b = ga.shape[0]
    ef = 0.25 * (r[0:b] + r[b:2 * b] + r[2 * b:3 * b] + r[3 * b:4 * b])
    feat_ref[...] = ef
    cm = jnp.dot(
        jnp.maximum(jnp.dot(ef, wc1_ref[...],
                            preferred_element_type=jnp.float32) + bc1_ref[...], 0.0),
        wc2_ref[...], preferred_element_type=jnp.float32)
    trans = jnp.clip(d * cm, -100.0, 100.0)
    small_ref[...] = jnp.concatenate(
        [trans, jnp.ones((b, 1), jnp.float32),
         jnp.zeros((b, HID - 3), jnp.float32)], axis=1)


def _edge_mlp(g2a, g2b, wx, we2, be2, wc1, bc1, wc2):
    ep = g2a.shape[0]
    grid = ep // EB
    return pl.pallas_call(
        _k3_body,
        grid=(grid,),
        in_specs=[
            pl.BlockSpec((EB, HID), lambda i: (i, 0)),
            pl.BlockSpec((EB, HID), lambda i: (i, 0)),
            pl.BlockSpec((8, HID), lambda i: (0, 0)),
            pl.BlockSpec((HID, HID), lambda i: (0, 0)),
            pl.BlockSpec((HID,), lambda i: (0,)),
            pl.BlockSpec((HID, HID), lambda i: (0, 0)),
            pl.BlockSpec((HID,), lambda i: (0,)),
            pl.BlockSpec((HID, 2), lambda i: (0, 0)),
        ],
        out_specs=[
            pl.BlockSpec((EB, HID), lambda i: (i, 0)),
            pl.BlockSpec((EB, HID), lambda i: (i, 0)),
        ],
        out_shape=[
            jax.ShapeDtypeStruct((ep, HID), jnp.float32),
            jax.ShapeDtypeStruct((ep, HID), jnp.float32),
        ],
    )(g2a, g2b, wx, we2, be2, wc1, bc1, wc2)


def _k5_body(h_ref, c_ref, vmv_ref, aggf_ref, aggs_ref, wn1a_ref, wn1b_ref,
             bn1_ref, wn2_ref, bn2_ref, hout_ref, cout_ref):
    h = h_ref[...]
    agg = aggf_ref[...]
    s = aggs_ref[...]
    cnt = jnp.maximum(s[:, 2:3], 1.0)
    cout_ref[...] = c_ref[...] + s[:, 0:2] / cnt + vmv_ref[...]
    t = jnp.maximum(
        jnp.dot(h, wn1a_ref[...], preferred_element_type=jnp.float32)
        + jnp.dot(agg, wn1b_ref[...], preferred_element_type=jnp.float32)
        + bn1_ref[...], 0.0)
    hout_ref[...] = (h + jnp.dot(t, wn2_ref[...],
                                 preferred_element_type=jnp.float32) + bn2_ref[...])


def _node_finalize(h_p, c_p, vmv, aggf, aggs, wn1a, wn1b, bn1, wn2, bn2):
    grid = NP_ // NB
    return pl.pallas_call(
        _k5_body,
        grid=(grid,),
        in_specs=[
            pl.BlockSpec((NB, INF), lambda i: (i, 0)),
            pl.BlockSpec((NB, 2), lambda i: (i, 0)),
            pl.BlockSpec((NB, 2), lambda i: (i, 0)),
            pl.BlockSpec((NB, HID), lambda i: (i, 0)),
            pl.BlockSpec((NB, HID), lambda i: (i, 0)),
            pl.BlockSpec((INF, HID), lambda i: (0, 0)),
            pl.BlockSpec((HID, HID), lambda i: (0, 0)),
            pl.BlockSpec((HID,), lambda i: (0,)),
            pl.BlockSpec((HID, OUT), lambda i: (0, 0)),
            pl.BlockSpec((OUT,), lambda i: (0,)),
        ],
        out_specs=[
            pl.BlockSpec((NB, OUT), lambda i: (i, 0)),
            pl.BlockSpec((NB, 2), lambda i: (i, 0)),
        ],
        out_shape=[
            jax.ShapeDtypeStruct((NP_, OUT), jnp.float32),
            jax.ShapeDtypeStruct((NP_, 2), jnp.float32),
        ],
    )(h_p, c_p, vmv, aggf, aggs, wn1a, wn1b, bn1, wn2, bn2)


def kernel(h, edge_index, coord, vel, We1, be1, We2, be2, Wn1, bn1, Wn2,
           bn2, Wc1, bc1, Wc2, Wv1, bv1, Wv2, bv2):
    h_p = jnp.pad(h, ((0, NP_ - N), (0, 0)))
    c_p = jnp.pad(coord, ((0, NP_ - N), (0, 0)))
    v_p = jnp.pad(vel, ((0, NP_ - N), (0, 0)))
    row = jnp.pad(edge_index[0], (0, EP_ - E), constant_values=PAD_IDX)
    col = jnp.pad(edge_index[1], (0, EP_ - E), constant_values=PAD_IDX)

    wa = We1[0:INF]
    wb = We1[INF:2 * INF]
    wx = jnp.concatenate([We1[2 * INF:2 * INF + 7], be1[None, :]], axis=0)

    ta, tb, vmv = _node_precompute(h_p, c_p, v_p, wa, wb, Wv1, bv1, Wv2, bv2)

    row2d = row.reshape(NCHUNK, CHUNK)
    col2d = col.reshape(NCHUNK, CHUNK)
    ta_p = _pack_bf16(ta)
    tb_p = _pack_bf16(tb)
    zf = jnp.zeros((NP_, HID), jnp.float32)

    nslice = 5
    hc = NCHUNK // nslice
    aggf, aggs = zf, zf
    for hidx in range(nslice):
        r2 = row2d[hidx * hc:(hidx + 1) * hc]
        c2 = col2d[hidx * hc:(hidx + 1) * hc]
        g2a, g2b = _sc_gather(ta_p, tb_p, r2, c2)
        feat, small = _edge_mlp(g2a, g2b, wx, We2, be2, Wc1, bc1, Wc2)
        aggf, aggs = _sc_scatter(r2, feat, small, aggf, aggs)

    hout, cout = _node_finalize(h_p, c_p, vmv, aggf, aggs, Wn1[0:INF],
                                Wn1[INF:], bn1, Wn2, bn2)
    return hout[:N], cout[:N]
